# baseline (speedup 1.0000x reference)
import functools

import jax
import jax.numpy as jnp
from jax import lax
from jax.experimental import pallas as pl
from jax.experimental.pallas import tpu as pltpu
from jax.experimental.pallas import tpu_sc as plsc

N = 10000
E = 320000
F = 128

NUM_WORKERS = 32
GCH = 80

NPT = 320
NPAD = NUM_WORKERS * NPT
CAP = 16384
LCAP = CAP // 16
CAP2 = CAP + 32
IC = 2000
MAXB = 10
SBPW = 16 * MAXB
SROWS = SBPW * GCH
RTOT = NUM_WORKERS * SROWS


def _mesh():
    return plsc.VectorSubcoreMesh(core_axis_name="c", subcore_axis_name="s")


def _wid():
    return lax.axis_index("s") * 2 + lax.axis_index("c")


def _tc_proj_body(x_ref, wt1_ref, bt1_ref, wg1_ref, bg1_ref,
                  pt_ref, qt_ref, pg_ref, qg_ref):
    x = x_ref[...]
    wta = wt1_ref[:F, :]
    wtb = wt1_ref[F:, :]
    wga = wg1_ref[:F, :]
    wgb = wg1_ref[F:, :]
    dot = functools.partial(jnp.dot, preferred_element_type=jnp.float32)
    pt_ref[...] = dot(x, wta - wtb)
    qt_ref[...] = dot(x, wtb) + bt1_ref[...]
    pg_ref[...] = dot(x, wga - wgb)
    qg_ref[...] = dot(x, wgb) + bg1_ref[...]


def _tc_proj(x, Wt1, bt1, Wg1, bg1):
    BN = 1000
    grid = (N // BN,)
    blk = pl.BlockSpec((BN, F), lambda i: (i, 0))
    full2 = pl.BlockSpec((2 * F, F), lambda i: (0, 0))
    bias = pl.BlockSpec((1, F), lambda i: (0, 0))
    out = jax.ShapeDtypeStruct((N, F), jnp.float32)
    return pl.pallas_call(
        _tc_proj_body,
        grid=grid,
        in_specs=[blk, full2, bias, full2, bias],
        out_specs=[blk, blk, blk, blk],
        out_shape=[out, out, out, out],
    )(x, Wt1, bt1.reshape(1, F), Wg1, bg1.reshape(1, F))


def _tc_edge_body(s_ref, w2_ref, b2_ref, h_ref):
    s = jnp.maximum(s_ref[...], 0.0)
    h = jnp.dot(s, w2_ref[...], preferred_element_type=jnp.float32)
    h_ref[...] = jnp.maximum(h + b2_ref[...], 0.0)


def _tc_edge(S, W2, b2):
    BS = 2048
    grid = (RTOT // BS,)
    blk = pl.BlockSpec((BS, F), lambda i: (i, 0))
    return pl.pallas_call(
        _tc_edge_body,
        grid=grid,
        in_specs=[blk,
                  pl.BlockSpec((F, F), lambda i: (0, 0)),
                  pl.BlockSpec((1, F), lambda i: (0, 0))],
        out_specs=blk,
        out_shape=jax.ShapeDtypeStruct((RTOT, F), jnp.float32),
    )(S, W2, b2.reshape(1, F))


def _tc_final_body(pt_ref, pg_ref, wf_ref, bf_ref, o_ref):
    dot = functools.partial(jnp.dot, preferred_element_type=jnp.float32)
    acc = dot(pt_ref[...], wf_ref[:F, :]) + dot(pg_ref[...], wf_ref[F:, :])
    o_ref[...] = jnp.maximum(acc + bf_ref[...], 0.0)


def _tc_final(pt, pg, Wf, bf):
    BN = 1000
    grid = (N // BN,)
    blk = pl.BlockSpec((BN, F), lambda i: (i, 0))
    return pl.pallas_call(
        _tc_final_body,
        grid=grid,
        in_specs=[blk, blk,
                  pl.BlockSpec((2 * F, F), lambda i: (0, 0)),
                  pl.BlockSpec((1, F), lambda i: (0, 0))],
        out_specs=blk,
        out_shape=jax.ShapeDtypeStruct((N, F), jnp.float32),
    )(pt, pg, Wf, bf.reshape(1, F))


def _sc_sg_body(p_hbm, q_hbm, ii_hbm, jj_hbm,
                s_hbm, nls_hbm, cnt_hbm,
                idxb0, idxb1, jdxb0, jdxb1, jvals, vals, nls, boff, cntb,
                rb0, rb1, rb2, rb3,
                sm0, sm1, sm2, sm3, sem2):
    wid = _wid()
    lo = wid * NPT
    hi = lo + NPT
    iota16 = lax.broadcasted_iota(jnp.int32, (16,), 0)
    lanebase = iota16 * LCAP
    zeros16i = jnp.zeros((16,), jnp.int32)
    dummy16i = jnp.full((16,), NPT * F, jnp.int32)

    def initi(v, c):
        for u in range(8):
            jvals[pl.ds(v * 128 + u * 16, 16)] = zeros16i
            vals[pl.ds(v * 128 + u * 16, 16)] = zeros16i
            nls[pl.ds(v * 128 + u * 16, 16)] = dummy16i
        return c
    lax.fori_loop(0, CAP2 // 128, initi, 0)
    for u in range((CAP2 % 128) // 16):
        jvals[pl.ds(CAP2 - CAP2 % 128 + u * 16, 16)] = zeros16i
        vals[pl.ds(CAP2 - CAP2 % 128 + u * 16, 16)] = zeros16i
        nls[pl.ds(CAP2 - CAP2 % 128 + u * 16, 16)] = dummy16i

    nchunk = E // IC
    pltpu.async_copy(ii_hbm.at[pl.ds(0, IC)], idxb0, sem2)
    pltpu.async_copy(jj_hbm.at[pl.ds(0, IC)], jdxb0, sem2)

    def scan_chunk(c, buf, jbuf, cursors):
        def vec(v, cur):
            for u in range(25):
                va = buf[pl.ds(v * 400 + u * 16, 16)]
                ja = jbuf[pl.ds(v * 400 + u * 16, 16)]
                m = (va >= lo) & (va < hi)
                pos = lanebase + cur
                plsc.store_scatter(jvals, [pos], ja, mask=m)
                plsc.store_scatter(vals, [pos], va, mask=m)
                plsc.store_scatter(nls, [pos], (va - lo) * F, mask=m)
                cur = jnp.minimum(cur + m.astype(jnp.int32), LCAP - 1)
            return cur

        return lax.fori_loop(0, IC // 400, vec, cursors)

    def pair(p, cursors):
        c0 = 2 * p
        pltpu.make_async_copy(ii_hbm.at[pl.ds(0, IC)], idxb0, sem2).wait()
        pltpu.make_async_copy(jj_hbm.at[pl.ds(0, IC)], jdxb0, sem2).wait()
        pltpu.async_copy(ii_hbm.at[pl.ds((c0 + 1) * IC, IC)], idxb1, sem2)
        pltpu.async_copy(jj_hbm.at[pl.ds((c0 + 1) * IC, IC)], jdxb1, sem2)
        cursors = scan_chunk(c0, idxb0, jdxb0, cursors)
        pltpu.make_async_copy(ii_hbm.at[pl.ds(0, IC)], idxb1, sem2).wait()
        pltpu.make_async_copy(jj_hbm.at[pl.ds(0, IC)], jdxb1, sem2).wait()

        @pl.when(c0 + 2 < nchunk)
        def _():
            pltpu.async_copy(ii_hbm.at[pl.ds((c0 + 2) * IC, IC)], idxb0, sem2)
            pltpu.async_copy(jj_hbm.at[pl.ds((c0 + 2) * IC, IC)], jdxb0, sem2)

        return scan_chunk(c0 + 1, idxb1, jdxb1, cursors)

    counts = lax.fori_loop(0, nchunk // 2, pair, zeros16i)

    nblks = jnp.minimum((counts + (GCH - 1)) // GCH, MAXB)
    cntb[pl.ds(0, 16)] = nblks

    def mkboff(l, cursor):
        nblk_l = cntb[pl.ds(l, 16)][0]
        offs = l * LCAP + iota16 * GCH
        plsc.store_scatter(boff, [cursor + iota16], offs,
                           mask=iota16 < nblk_l)
        return cursor + nblk_l

    nbt = lax.fori_loop(0, 16, mkboff, jnp.int32(0))
    pltpu.sync_copy(cntb.at[pl.ds(0, 16)], cnt_hbm.at[pl.ds(wid * 16, 16)])
    pltpu.sync_copy(nls.at[pl.ds(0, CAP2)],
                    nls_hbm.at[pl.ds(wid * CAP2, CAP2)])

    def getoff(t):
        return pl.multiple_of(boff[pl.ds(t, 16)][0], 8)

    srow0 = wid * SROWS

    def issue_p(t, rb, sm):
        off = getoff(t)
        pltpu.async_copy(p_hbm.at[vals.at[pl.ds(off, GCH)]], rb, sm)

    def run_block(t, rb, sm):
        off = getoff(t)
        pltpu.make_async_copy(p_hbm.at[vals.at[pl.ds(0, GCH)]], rb,
                              sm).wait()
        pltpu.async_copy(q_hbm.at[jvals.at[pl.ds(off, GCH)]], rb, sm,
                         add=True)
        pltpu.make_async_copy(p_hbm.at[vals.at[pl.ds(0, GCH)]], rb,
                              sm).wait()
        pltpu.async_copy(rb, s_hbm.at[pl.ds(srow0 + t * GCH, GCH)], sm)
        pltpu.make_async_copy(rb, s_hbm.at[pl.ds(srow0, GCH)], sm).wait()

    slots = list(zip((rb0, rb1, rb2, rb3), (sm0, sm1, sm2, sm3)))
    for s, (rb, sm) in enumerate(slots):
        @pl.when(s < nbt)
        def _(s=s, rb=rb, sm=sm):
            issue_p(s, rb, sm)

    def quad(q, carry):
        t0 = 4 * q
        for s, (rb, sm) in enumerate(slots):
            t = t0 + s

            @pl.when(t < nbt)
            def _(t=t, rb=rb, sm=sm):
                run_block(t, rb, sm)

                @pl.when(t + 4 < nbt)
                def _():
                    issue_p(t + 4, rb, sm)
        return carry

    lax.fori_loop(0, (nbt + 3) // 4, quad, 0)


def _sc_sg(P, Q, ii, jj):
    k = functools.partial(
        pl.kernel,
        out_type=[
            jax.ShapeDtypeStruct((RTOT, F), jnp.float32),
            jax.ShapeDtypeStruct((NUM_WORKERS * CAP2,), jnp.int32),
            jax.ShapeDtypeStruct((NUM_WORKERS * 16,), jnp.int32),
        ],
        mesh=_mesh(),
        compiler_params=pltpu.CompilerParams(needs_layout_passes=False),
        scratch_types=[
            pltpu.VMEM((IC,), jnp.int32),
            pltpu.VMEM((IC,), jnp.int32),
            pltpu.VMEM((IC,), jnp.int32),
            pltpu.VMEM((IC,), jnp.int32),
            pltpu.VMEM((CAP2,), jnp.int32),
            pltpu.VMEM((CAP2,), jnp.int32),
            pltpu.VMEM((CAP2,), jnp.int32),
            pltpu.VMEM((224,), jnp.int32),
            pltpu.VMEM((32,), jnp.int32),
            pltpu.VMEM((GCH, F), jnp.float32),
            pltpu.VMEM((GCH, F), jnp.float32),
            pltpu.VMEM((GCH, F), jnp.float32),
            pltpu.VMEM((GCH, F), jnp.float32),
            pltpu.SemaphoreType.DMA,
            pltpu.SemaphoreType.DMA,
            pltpu.SemaphoreType.DMA,
            pltpu.SemaphoreType.DMA,
            pltpu.SemaphoreType.DMA,
        ],
    )(_sc_sg_body)
    return k(P, Q, ii, jj)


def _sc_ls_body(h_hbm, nls_hbm, cnt_hbm, pool_hbm,
                nls, boff, cntb, pooled, ra, rb, sma, smb):
    wid = _wid()
    lo = wid * NPT
    iota16 = lax.broadcasted_iota(jnp.int32, (16,), 0)
    zeros16f = jnp.zeros((16,), jnp.float32)

    pltpu.sync_copy(cnt_hbm.at[pl.ds(wid * 16, 16)], cntb.at[pl.ds(0, 16)])
    pltpu.sync_copy(nls_hbm.at[pl.ds(wid * CAP2, CAP2)],
                    nls.at[pl.ds(0, CAP2)])

    def initp(v, c):
        for u in range(8):
            pooled[pl.ds(v * 128 + u * 16, 16)] = zeros16f
        return c
    lax.fori_loop(0, (NPT * F + 128) // 128, initp, 0)

    def mkboff(l, cursor):
        nblk_l = cntb[pl.ds(l, 16)][0]
        offs = l * LCAP + iota16 * GCH
        plsc.store_scatter(boff, [cursor + iota16], offs,
                           mask=iota16 < nblk_l)
        return cursor + nblk_l

    nbt = lax.fori_loop(0, 16, mkboff, jnp.int32(0))

    def getoff(t):
        return pl.multiple_of(boff[pl.ds(t, 16)][0], 8)

    srow0 = wid * SROWS

    def issue(t, buf, sm):
        pltpu.async_copy(h_hbm.at[pl.ds(srow0 + t * GCH, GCH)], buf, sm)

    def wdma(buf, sm):
        pltpu.make_async_copy(h_hbm.at[pl.ds(srow0, GCH)], buf, sm).wait()

    def compute(t, buf):
        off = getoff(t)

        def grp(g, c2):
            nbv = nls[pl.ds(off + g * 16, 16)]
            for r in range(16):
                nb = nbv[r]
                hvs = [buf[g * 16 + r, pl.ds(kk * 16, 16)]
                       for kk in range(F // 16)]
                pvs = [pooled[pl.ds(nb + kk * 16, 16)]
                       for kk in range(F // 16)]
                for kk in range(F // 16):
                    pooled[pl.ds(nb + kk * 16, 16)] = jnp.maximum(
                        pvs[kk], hvs[kk])
            return c2

        lax.fori_loop(0, GCH // 16, grp, 0)

    for s, (buf, sm) in enumerate(((ra, sma), (rb, smb))):
        @pl.when(s < nbt)
        def _(s=s, buf=buf, sm=sm):
            issue(s, buf, sm)

    def duo(q, carry):
        t0 = 2 * q
        for s, (buf, sm) in enumerate(((ra, sma), (rb, smb))):
            t = t0 + s

            @pl.when(t < nbt)
            def _(t=t, buf=buf, sm=sm):
                wdma(buf, sm)
                compute(t, buf)

                @pl.when(t + 2 < nbt)
                def _():
                    issue(t + 2, buf, sm)
        return carry

    lax.fori_loop(0, (nbt + 1) // 2, duo, 0)

    pltpu.sync_copy(pooled.at[pl.ds(0, NPT * F)],
                    pool_hbm.at[pl.ds(lo * F, NPT * F)])


def _sc_ls(H, nlsq, cnts):
    k = functools.partial(
        pl.kernel,
        out_type=jax.ShapeDtypeStruct((NPAD * F,), jnp.float32),
        mesh=_mesh(),
        compiler_params=pltpu.CompilerParams(needs_layout_passes=False),
        scratch_types=[
            pltpu.VMEM((CAP2,), jnp.int32),
            pltpu.VMEM((224,), jnp.int32),
            pltpu.VMEM((32,), jnp.int32),
            pltpu.VMEM((NPT * F + 128,), jnp.float32),
            pltpu.VMEM((GCH, F), jnp.float32),
            pltpu.VMEM((GCH, F), jnp.float32),
            pltpu.SemaphoreType.DMA,
            pltpu.SemaphoreType.DMA,
        ],
    )(_sc_ls_body)
    return k(H, nlsq, cnts).reshape(NPAD, F)


def kernel(x, edge_index_topo, edge_index_geo,
           Wt1, bt1, Wt2, bt2, Wg1, bg1, Wg2, bg2, Wf, bf):
    ii_t = edge_index_topo[0]
    jj_t = edge_index_topo[1]
    ii_g = edge_index_geo[0]
    jj_g = edge_index_geo[1]

    Pt, Qt, Pg, Qg = _tc_proj(x, Wt1, bt1, Wg1, bg1)

    St, nl_t, cn_t = _sc_sg(Pt, Qt, ii_t, jj_t)
    Sg, nl_g, cn_g = _sc_sg(Pg, Qg, ii_g, jj_g)

    Ht = _tc_edge(St, Wt2, bt2)
    Hg = _tc_edge(Sg, Wg2, bg2)

    pt = _sc_ls(Ht, nl_t, cn_t)
    pg = _sc_ls(Hg, nl_g, cn_g)

    return _tc_final(pt, pg, Wf, bf)


# speedup vs baseline: 1.2869x; 1.2869x over previous
import functools

import jax
import jax.numpy as jnp
from jax import lax
from jax.experimental import pallas as pl
from jax.experimental.pallas import tpu as pltpu
from jax.experimental.pallas import tpu_sc as plsc

N = 10000
E = 320000
F = 128

NUM_WORKERS = 32
EPW = E // NUM_WORKERS
GCH = 80

NPT = 320
NPAD = NUM_WORKERS * NPT
CAP = 16384
IC = 2000


def _mesh():
    return plsc.VectorSubcoreMesh(core_axis_name="c", subcore_axis_name="s")


def _wid():
    return lax.axis_index("s") * 2 + lax.axis_index("c")


def _tc_proj_body(x_ref, wt1_ref, bt1_ref, wg1_ref, bg1_ref,
                  pt_ref, qt_ref, pg_ref, qg_ref):
    x = x_ref[...]
    wta = wt1_ref[:F, :]
    wtb = wt1_ref[F:, :]
    wga = wg1_ref[:F, :]
    wgb = wg1_ref[F:, :]
    dot = functools.partial(jnp.dot, preferred_element_type=jnp.float32)
    pt_ref[...] = dot(x, wta - wtb)
    qt_ref[...] = dot(x, wtb) + bt1_ref[...]
    pg_ref[...] = dot(x, wga - wgb)
    qg_ref[...] = dot(x, wgb) + bg1_ref[...]


def _tc_proj(x, Wt1, bt1, Wg1, bg1):
    BN = 1000
    grid = (N // BN,)
    blk = pl.BlockSpec((BN, F), lambda i: (i, 0))
    full2 = pl.BlockSpec((2 * F, F), lambda i: (0, 0))
    bias = pl.BlockSpec((1, F), lambda i: (0, 0))
    out = jax.ShapeDtypeStruct((N, F), jnp.float32)
    return pl.pallas_call(
        _tc_proj_body,
        grid=grid,
        in_specs=[blk, full2, bias, full2, bias],
        out_specs=[blk, blk, blk, blk],
        out_shape=[out, out, out, out],
    )(x, Wt1, bt1.reshape(1, F), Wg1, bg1.reshape(1, F))


def _tc_edge_body(s_ref, w2_ref, b2_ref, h_ref):
    s = jnp.maximum(s_ref[...], 0.0)
    h = jnp.dot(s, w2_ref[...], preferred_element_type=jnp.float32)
    h_ref[...] = jnp.maximum(h + b2_ref[...], 0.0)


def _tc_edge(S, W2, b2):
    BS = 2000
    grid = (E // BS,)
    blk = pl.BlockSpec((BS, F), lambda i: (i, 0))
    return pl.pallas_call(
        _tc_edge_body,
        grid=grid,
        in_specs=[blk,
                  pl.BlockSpec((F, F), lambda i: (0, 0)),
                  pl.BlockSpec((1, F), lambda i: (0, 0))],
        out_specs=blk,
        out_shape=jax.ShapeDtypeStruct((E, F), jnp.float32),
    )(S, W2, b2.reshape(1, F))


def _tc_final_body(pt_ref, pg_ref, wf_ref, bf_ref, o_ref):
    dot = functools.partial(jnp.dot, preferred_element_type=jnp.float32)
    acc = dot(pt_ref[...], wf_ref[:F, :]) + dot(pg_ref[...], wf_ref[F:, :])
    o_ref[...] = jnp.maximum(acc + bf_ref[...], 0.0)


def _tc_final(pt, pg, Wf, bf):
    BN = 1000
    grid = (N // BN,)
    blk = pl.BlockSpec((BN, F), lambda i: (i, 0))
    return pl.pallas_call(
        _tc_final_body,
        grid=grid,
        in_specs=[blk, blk,
                  pl.BlockSpec((2 * F, F), lambda i: (0, 0)),
                  pl.BlockSpec((1, F), lambda i: (0, 0))],
        out_specs=blk,
        out_shape=jax.ShapeDtypeStruct((N, F), jnp.float32),
    )(pt, pg, Wf, bf.reshape(1, F))


GB = 128
NFB = EPW // GB


def _sc_gather_body(p_hbm, q_hbm, ii_hbm, jj_hbm, s_hbm, iv, jv, bufa, bufb,
                    sema, semb):
    wid = _wid()
    base = wid * EPW
    pltpu.sync_copy(ii_hbm.at[pl.ds(base, EPW)], iv)
    pltpu.sync_copy(jj_hbm.at[pl.ds(base, EPW)], jv)

    def gp(buf, sem, g):
        return pltpu.async_copy(p_hbm.at[iv.at[pl.ds(g * GB, GB)]], buf, sem)

    def gq(buf, sem, g):
        return pltpu.async_copy(q_hbm.at[jv.at[pl.ds(g * GB, GB)]], buf, sem,
                                add=True)

    def st(buf, sem, g):
        return pltpu.async_copy(buf, s_hbm.at[pl.ds(base + g * GB, GB)], sem)

    def wp(buf, sem, g):
        pltpu.make_async_copy(p_hbm.at[iv.at[pl.ds(g * GB, GB)]], buf,
                              sem).wait()

    def ws(buf, sem, g):
        pltpu.make_async_copy(buf, s_hbm.at[pl.ds(base + g * GB, GB)],
                              sem).wait()

    gp(bufa, sema, 0)

    def body(p, carry):
        g0 = 2 * p
        g1 = g0 + 1
        gp(bufb, semb, g1)
        wp(bufa, sema, g0)
        gq(bufa, sema, g0)
        wp(bufa, sema, g0)
        st(bufa, sema, g0)
        wp(bufb, semb, g1)
        gq(bufb, semb, g1)
        ws(bufa, sema, g0)

        @pl.when(g0 + 2 < NFB)
        def _():
            gp(bufa, sema, g0 + 2)

        wp(bufb, semb, g1)
        st(bufb, semb, g1)
        ws(bufb, semb, g1)
        return carry

    lax.fori_loop(0, NFB // 2, body, 0)

    tail = EPW - NFB * GB
    if tail:
        toff = NFB * GB
        ta = bufa.at[pl.ds(0, tail), :]
        pltpu.async_copy(p_hbm.at[iv.at[pl.ds(toff, tail)]], ta, sema).wait()
        pltpu.async_copy(q_hbm.at[jv.at[pl.ds(toff, tail)]], ta, sema,
                         add=True).wait()
        pltpu.sync_copy(ta, s_hbm.at[pl.ds(base + toff, tail)])


def _sc_gather(P, Q, ii, jj):
    k = functools.partial(
        pl.kernel,
        out_type=jax.ShapeDtypeStruct((E, F), jnp.float32),
        mesh=_mesh(),
        compiler_params=pltpu.CompilerParams(needs_layout_passes=False),
        scratch_types=[
            pltpu.VMEM((EPW,), jnp.int32),
            pltpu.VMEM((EPW,), jnp.int32),
            pltpu.VMEM((GB, F), jnp.float32),
            pltpu.VMEM((GB, F), jnp.float32),
            pltpu.SemaphoreType.DMA,
            pltpu.SemaphoreType.DMA,
        ],
    )(_sc_gather_body)
    return k(P, Q, ii, jj)


LCAP = CAP // 16
CAP2 = CAP + 32


def _sc_scatter_body(h_hbm, ii_hbm, pool_hbm, idxb0, idxb1, eids, nls, pooled,
                     rows, rowsb, rowsc, rowsd, cntb, boff, sem, semb, semc,
                     semd, sem2):
    wid = _wid()
    lo = wid * NPT
    hi = lo + NPT
    iota16 = lax.broadcasted_iota(jnp.int32, (16,), 0)
    lanebase = iota16 * LCAP
    zeros16i = jnp.zeros((16,), jnp.int32)
    zeros16f = jnp.zeros((16,), jnp.float32)
    dummy16i = jnp.full((16,), NPT * F, jnp.int32)

    def initi(v, c):
        for u in range(8):
            eids[pl.ds(v * 128 + u * 16, 16)] = zeros16i
            nls[pl.ds(v * 128 + u * 16, 16)] = dummy16i
        return c
    lax.fori_loop(0, CAP2 // 128, initi, 0)
    for u in range((CAP2 % 128) // 16):
        eids[pl.ds(CAP2 - CAP2 % 128 + u * 16, 16)] = zeros16i
        nls[pl.ds(CAP2 - CAP2 % 128 + u * 16, 16)] = dummy16i

    def initp(v, c):
        for u in range(8):
            pooled[pl.ds(v * 128 + u * 16, 16)] = zeros16f
        return c
    lax.fori_loop(0, (NPT * F + 128) // 128, initp, 0)

    nchunk = E // IC
    pltpu.async_copy(ii_hbm.at[pl.ds(0, IC)], idxb0, sem2)

    def scan_chunk(c, buf, cursors):
        def vec(v, cur):
            for u in range(25):
                vals = buf[pl.ds(v * 400 + u * 16, 16)]
                m = (vals >= lo) & (vals < hi)
                eid = c * IC + v * 400 + u * 16 + iota16
                pos = lanebase + cur
                plsc.store_scatter(eids, [pos], eid, mask=m)
                plsc.store_scatter(nls, [pos], (vals - lo) * F, mask=m)
                cur = jnp.minimum(cur + m.astype(jnp.int32), LCAP - 1)
            return cur

        return lax.fori_loop(0, IC // 400, vec, cursors)

    def pair(p, cursors):
        c0 = 2 * p
        pltpu.make_async_copy(ii_hbm.at[pl.ds(0, IC)], idxb0, sem2).wait()
        pltpu.async_copy(ii_hbm.at[pl.ds((c0 + 1) * IC, IC)], idxb1, sem2)
        cursors = scan_chunk(c0, idxb0, cursors)
        pltpu.make_async_copy(ii_hbm.at[pl.ds(0, IC)], idxb1, sem2).wait()

        @pl.when(c0 + 2 < nchunk)
        def _():
            pltpu.async_copy(ii_hbm.at[pl.ds((c0 + 2) * IC, IC)], idxb0, sem2)

        return scan_chunk(c0 + 1, idxb1, cursors)

    counts = lax.fori_loop(0, nchunk // 2, pair, zeros16i)

    nblks = (counts + (GCH - 1)) // GCH
    cntb[pl.ds(0, 16)] = nblks

    def mkboff(l, cursor):
        nblk_l = cntb[pl.ds(l, 16)][0]
        offs = l * LCAP + iota16 * GCH
        plsc.store_scatter(boff, [cursor + iota16], offs,
                           mask=iota16 < nblk_l)
        return cursor + nblk_l

    nbt = lax.fori_loop(0, 16, mkboff, jnp.int32(0))

    def getoff(t):
        return pl.multiple_of(boff[pl.ds(t, 16)][0], 8)

    def issue(t, buf, sm):
        pltpu.async_copy(h_hbm.at[eids.at[pl.ds(getoff(t), GCH)]], buf, sm)

    def wdma(buf, sm):
        pltpu.make_async_copy(h_hbm.at[eids.at[pl.ds(0, GCH)]], buf,
                              sm).wait()

    def compute(t, buf):
        off = getoff(t)

        def grp(g, c2):
            nbv = nls[pl.ds(off + g * 16, 16)]
            for r in range(16):
                nb = nbv[r]
                hvs = [buf[g * 16 + r, pl.ds(kk * 16, 16)]
                       for kk in range(F // 16)]
                pvs = [pooled[pl.ds(nb + kk * 16, 16)]
                       for kk in range(F // 16)]
                for kk in range(F // 16):
                    pooled[pl.ds(nb + kk * 16, 16)] = jnp.maximum(
                        pvs[kk], hvs[kk])
            return c2

        lax.fori_loop(0, GCH // 16, grp, 0)

    slots = list(zip((rows, rowsb, rowsc, rowsd), (sem, semb, semc, semd)))
    for s, (buf, sm) in enumerate(slots):
        @pl.when(s < nbt)
        def _(buf=buf, sm=sm, s=s):
            issue(s, buf, sm)

    def quad(q, carry):
        t0 = 4 * q
        for s, (buf, sm) in enumerate(slots):
            t = t0 + s

            @pl.when(t < nbt)
            def _(t=t, buf=buf, sm=sm):
                wdma(buf, sm)
                compute(t, buf)

                @pl.when(t + 4 < nbt)
                def _():
                    issue(t + 4, buf, sm)
        return carry

    lax.fori_loop(0, (nbt + 3) // 4, quad, 0)

    pltpu.sync_copy(pooled.at[pl.ds(0, NPT * F)],
                    pool_hbm.at[pl.ds(lo * F, NPT * F)])


def _sc_scatter(H, ii):
    k = functools.partial(
        pl.kernel,
        out_type=jax.ShapeDtypeStruct((NPAD * F,), jnp.float32),
        mesh=_mesh(),
        compiler_params=pltpu.CompilerParams(needs_layout_passes=False),
        scratch_types=[
            pltpu.VMEM((IC,), jnp.int32),
            pltpu.VMEM((IC,), jnp.int32),
            pltpu.VMEM((CAP2,), jnp.int32),
            pltpu.VMEM((CAP2,), jnp.int32),
            pltpu.VMEM((NPT * F + 128,), jnp.float32),
            pltpu.VMEM((GCH, F), jnp.float32),
            pltpu.VMEM((GCH, F), jnp.float32),
            pltpu.VMEM((GCH, F), jnp.float32),
            pltpu.VMEM((GCH, F), jnp.float32),
            pltpu.VMEM((32,), jnp.int32),
            pltpu.VMEM((224,), jnp.int32),
            pltpu.SemaphoreType.DMA,
            pltpu.SemaphoreType.DMA,
            pltpu.SemaphoreType.DMA,
            pltpu.SemaphoreType.DMA,
            pltpu.SemaphoreType.DMA,
        ],
    )(_sc_scatter_body)
    return k(H, ii).reshape(NPAD, F)


def kernel(x, edge_index_topo, edge_index_geo,
           Wt1, bt1, Wt2, bt2, Wg1, bg1, Wg2, bg2, Wf, bf):
    ii_t = edge_index_topo[0]
    jj_t = edge_index_topo[1]
    ii_g = edge_index_geo[0]
    jj_g = edge_index_geo[1]

    Pt, Qt, Pg, Qg = _tc_proj(x, Wt1, bt1, Wg1, bg1)

    St = _sc_gather(Pt, Qt, ii_t, jj_t)
    Sg = _sc_gather(Pg, Qg, ii_g, jj_g)

    Ht = _tc_edge(St, Wt2, bt2)
    Hg = _tc_edge(Sg, Wg2, bg2)

    pt = _sc_scatter(Ht, ii_t)
    pg = _sc_scatter(Hg, ii_g)

    return _tc_final(pt, pg, Wf, bf)
